# baseline (speedup 1.0000x reference)
import functools

import jax
import jax.numpy as jnp
from jax import lax
from jax.experimental import pallas as pl
from jax.experimental.pallas import tpu as pltpu
from jax.experimental.pallas import tpu_sc as plsc

_NUM_CORES = 2
_NUM_SUBCORES = 16
_NUM_WORKERS = _NUM_CORES * _NUM_SUBCORES

_BB = 128
_L = 16
_NBUF = 4


@functools.lru_cache(maxsize=None)
def _make_gather(V, D, B, H):
    assert D % _L == 0 and B % (_NUM_WORKERS * _BB) == 0
    td_n = D // 8
    tb_per_w = B // (_NUM_WORKERS * _BB)
    b_per_w = tb_per_w * _BB
    n_blocks = H * tb_per_w
    assert n_blocks % _NBUF == 0

    mesh = plsc.VectorSubcoreMesh(
        core_axis_name="c",
        subcore_axis_name="s",
        num_cores=_NUM_CORES,
        num_subcores=_NUM_SUBCORES,
    )

    @functools.partial(
        pl.kernel,
        mesh=mesh,
        out_type=jax.ShapeDtypeStruct((H, td_n, B // _BB, 8 * _BB), jnp.float32),
        scratch_types=[
            pltpu.VMEM((H, b_per_w), jnp.int32),
            [pltpu.VMEM((_BB, D), jnp.float32) for _ in range(_NBUF)],
            [pltpu.VMEM((td_n, 8 * _BB), jnp.float32) for _ in range(_NBUF)],
            [pltpu.SemaphoreType.DMA for _ in range(_NBUF)],
            [pltpu.SemaphoreType.DMA for _ in range(_NBUF)],
        ],
        compiler_params=pltpu.CompilerParams(
            use_tc_tiling_on_sc=False, needs_layout_passes=False),
    )
    def gather_kernel(xt_hbm, table_hbm, out_hbm, idx_all, rows, touts, gsems, wsems):
        wid = lax.axis_index("s") * _NUM_CORES + lax.axis_index("c")
        tb_base = wid * tb_per_w

        pltpu.sync_copy(xt_hbm.at[:, pl.ds(wid * b_per_w, b_per_w)], idx_all)

        def coords(i):
            return i // tb_per_w, i % tb_per_w

        def gather_desc(i, p):
            h, tbl = coords(i)
            return pltpu.make_async_copy(
                table_hbm.at[idx_all.at[h, pl.ds(tbl * _BB, _BB)]],
                rows[p],
                gsems[p],
            )

        def wb_descs(i, p):
            h, tbl = coords(i)
            return [
                pltpu.make_async_copy(
                    touts[p],
                    out_hbm.at[h, :, tb_base + tbl],
                    wsems[p],
                )
            ]

        lane_d = [v * _L + lax.iota(jnp.int32, _L) for v in range(D // _L)]
        scatter_td = [d // 8 for d in lane_d]
        scatter_base = [(d % 8) * _BB for d in lane_d]

        def transpose_block(p):
            @plsc.parallel_loop(0, _BB, step=1, unroll=16)
            def _t(b):
                for v in range(D // _L):
                    vec = rows[p][b, pl.ds(v * _L, _L)]
                    plsc.store_scatter(
                        touts[p], [scatter_td[v], scatter_base[v] + b], vec)

        for p in range(_NBUF):
            gather_desc(jnp.int32(p), p).start()

        @pl.loop(0, n_blocks, step=_NBUF)
        def _blocks(i0):
            for p in range(_NBUF):
                i = i0 + p
                gather_desc(i, p).wait()

                @pl.when(i >= _NBUF)
                def _wait_prev_wb():
                    for d in wb_descs(i - _NBUF, p):
                        d.wait()

                transpose_block(p)
                for d in wb_descs(i, p):
                    d.start()

                @pl.when(i + _NBUF < n_blocks)
                def _next():
                    gather_desc(i + _NBUF, p).start()

        for p in range(_NBUF):
            for d in wb_descs(jnp.int32(n_blocks - _NBUF + p), p):
                d.wait()

    return gather_kernel


def kernel(x, table):
    batch, hist = x.shape
    vocab, dim = table.shape
    xt = x.T.astype(jnp.int32)
    out5 = _make_gather(vocab, dim, batch, hist)(xt, table)
    out5 = out5.reshape(hist, dim // 8, batch // _BB, 8, _BB)
    return out5.transpose(2, 4, 0, 1, 3).reshape(batch, hist, dim)


# speedup vs baseline: 1.0041x; 1.0041x over previous
import functools

import jax
import jax.numpy as jnp
from jax import lax
from jax.experimental import pallas as pl
from jax.experimental.pallas import tpu as pltpu
from jax.experimental.pallas import tpu_sc as plsc

_NUM_CORES = 2
_NUM_SUBCORES = 16
_NUM_WORKERS = _NUM_CORES * _NUM_SUBCORES

_BB = 128
_L = 16
_NBUF = 4


@functools.lru_cache(maxsize=None)
def _make_gather(V, D, B, H):
    assert D % _L == 0 and B % (_NUM_WORKERS * _BB) == 0
    td_n = D // 8
    tb_per_w = B // (_NUM_WORKERS * _BB)
    b_per_w = tb_per_w * _BB
    n_blocks = H * tb_per_w
    assert n_blocks % _NBUF == 0

    mesh = plsc.VectorSubcoreMesh(
        core_axis_name="c",
        subcore_axis_name="s",
        num_cores=_NUM_CORES,
        num_subcores=_NUM_SUBCORES,
    )

    @functools.partial(
        pl.kernel,
        mesh=mesh,
        out_type=jax.ShapeDtypeStruct((H, td_n, B // _BB, 8 * _BB), jnp.float32),
        scratch_types=[
            pltpu.VMEM((H, b_per_w), jnp.int32),
            [pltpu.VMEM((_BB, D), jnp.float32) for _ in range(_NBUF)],
            [pltpu.VMEM((td_n, 8 * _BB), jnp.float32) for _ in range(_NBUF)],
            [pltpu.SemaphoreType.DMA for _ in range(_NBUF)],
            [pltpu.SemaphoreType.DMA for _ in range(_NBUF)],
        ],
        compiler_params=pltpu.CompilerParams(
            use_tc_tiling_on_sc=False, needs_layout_passes=False),
    )
    def gather_kernel(xt_hbm, table_hbm, out_hbm, idx_all, rows, touts, gsems, wsems):
        wid = lax.axis_index("s") * _NUM_CORES + lax.axis_index("c")
        tb_base = wid * tb_per_w

        pltpu.sync_copy(xt_hbm.at[:, pl.ds(wid * b_per_w, b_per_w)], idx_all)

        def coords(i):
            return i // tb_per_w, i % tb_per_w

        def gather_desc(i, p):
            h, tbl = coords(i)
            return pltpu.make_async_copy(
                table_hbm.at[idx_all.at[h, pl.ds(tbl * _BB, _BB)]],
                rows[p],
                gsems[p],
            )

        def wb_descs(i, p):
            h, tbl = coords(i)
            return [
                pltpu.make_async_copy(
                    touts[p],
                    out_hbm.at[h, :, tb_base + tbl],
                    wsems[p],
                )
            ]

        lane_d = [v * _L + lax.iota(jnp.int32, _L) for v in range(D // _L)]
        scatter_td = [d // 8 for d in lane_d]
        scatter_base = [(d % 8) * _BB for d in lane_d]

        def transpose_block(p):
            @plsc.parallel_loop(0, _BB, step=1, unroll=8)
            def _t(b):
                for v in range(D // _L):
                    vec = rows[p][b, pl.ds(v * _L, _L)]
                    plsc.store_scatter(
                        touts[p], [scatter_td[v], scatter_base[v] + b], vec)

        for p in range(_NBUF):
            gather_desc(jnp.int32(p), p).start()

        @pl.loop(0, n_blocks, step=_NBUF)
        def _blocks(i0):
            for p in range(_NBUF):
                i = i0 + p
                gather_desc(i, p).wait()

                @pl.when(i >= _NBUF)
                def _wait_prev_wb():
                    for d in wb_descs(i - _NBUF, p):
                        d.wait()

                transpose_block(p)
                for d in wb_descs(i, p):
                    d.start()

                @pl.when(i + _NBUF < n_blocks)
                def _next():
                    gather_desc(i + _NBUF, p).start()

        for p in range(_NBUF):
            for d in wb_descs(jnp.int32(n_blocks - _NBUF + p), p):
                d.wait()

    return gather_kernel


def kernel(x, table):
    batch, hist = x.shape
    vocab, dim = table.shape
    xt = x.T.astype(jnp.int32)
    out5 = _make_gather(vocab, dim, batch, hist)(xt, table)
    out5 = out5.reshape(hist, dim // 8, batch // _BB, 8, _BB)
    return out5.transpose(2, 4, 0, 1, 3).reshape(batch, hist, dim)
